# baseline (speedup 1.0000x reference)
import functools

import jax
import jax.numpy as jnp
from jax import lax
from jax.experimental import pallas as pl
from jax.experimental.pallas import tpu as pltpu
from jax.experimental.pallas import tpu_sc as plsc

D = 1024
H = 16
DH = 64
NHID = 2048
E = 8
S = 2048
EPS = 1e-5

BT = 256
BT_SHIFT = 8
NB = S // BT + E - 1
SPAD = NB * BT

BQ = 512

NW = 32
BPW = S // NW


def _gelu_exact(x):
    z = x * 0.7071067811865476
    a = jnp.abs(z)
    t = 1.0 / (1.0 + 0.3275911 * a)
    poly = t * (0.254829592 + t * (-0.284496736 + t * (1.421413741
               + t * (-1.453152027 + t * 1.061405429))))
    erf_abs = 1.0 - poly * jnp.exp(-a * a)
    erf = jnp.sign(z) * erf_abs
    return 0.5 * x * (1.0 + erf)


def _ln(r, g, b):
    mu = jnp.mean(r, axis=1, keepdims=True)
    var = jnp.mean((r - mu) ** 2, axis=1, keepdims=True)
    return (r - mu) * lax.rsqrt(var + EPS) * g + b


def _dot_nt(a, b):
    return lax.dot_general(a, b, (((1,), (1,)), ((), ())),
                           preferred_element_type=jnp.float32)


def _dot_nn(a, b):
    return lax.dot_general(a, b, (((1,), (0,)), ((), ())),
                           preferred_element_type=jnp.float32)


def _qkv_body(x_ref, w_ref, b_ref, o_ref):
    o_ref[...] = _dot_nt(x_ref[...], w_ref[...]) + b_ref[...]


def _attn_body(q_ref, k_ref, v_ref, o_ref):
    q = (q_ref[0] * 0.125).astype(jnp.bfloat16)
    s = _dot_nt(q, k_ref[0].astype(jnp.bfloat16))
    m = jnp.max(s, axis=1, keepdims=True)
    p = jnp.exp(s - m)
    den = jnp.sum(p, axis=1, keepdims=True)
    o_ref[0] = _dot_nn(p.astype(jnp.bfloat16), v_ref[0].astype(jnp.bfloat16)) / den


def _post_attn_body(o_ref, x_ref, wo_ref, bo_ref, g1_ref, be1_ref,
                    wg_ref, bg_ref, y_ref, logits_ref):
    att = _dot_nt(o_ref[...], wo_ref[...]) + bo_ref[...]
    y = _ln(x_ref[...] + att, g1_ref[...], be1_ref[...])
    y_ref[...] = y
    logits_ref[...] = _dot_nt(y, wg_ref[...]) + bg_ref[...]


def _route_body(logits_ref, pos_ref, bexp_ref):
    logits = logits_ref[...]
    m = jnp.max(logits, axis=1, keepdims=True)
    iota_e = lax.broadcasted_iota(jnp.int32, (S, E), 1)
    idxm = jnp.where(logits == m, iota_e, E)
    idx = jnp.min(idxm, axis=1, keepdims=True)
    onehot = (iota_e == idx).astype(jnp.float32)

    r_io = lax.broadcasted_iota(jnp.int32, (S, S), 0)
    c_io = lax.broadcasted_iota(jnp.int32, (S, S), 1)
    tril = (r_io > c_io).astype(jnp.bfloat16)
    ranks_all = lax.dot_general(tril, onehot.astype(jnp.bfloat16),
                                (((1,), (0,)), ((), ())),
                                preferred_element_type=jnp.float32)

    counts = jnp.sum(onehot, axis=0, keepdims=True)
    counts_i = counts.astype(jnp.int32)
    pc = ((counts_i + (BT - 1)) >> BT_SHIFT) << BT_SHIFT
    e_r = lax.broadcasted_iota(jnp.int32, (E, E), 0)
    e_c = lax.broadcasted_iota(jnp.int32, (E, E), 1)
    tri8 = (e_r < e_c).astype(jnp.float32)
    off = _dot_nn(pc.astype(jnp.float32), tri8)
    off_i = off.astype(jnp.int32)

    pos_f = jnp.sum((ranks_all + off) * onehot, axis=1, keepdims=True)
    pos_ref[...] = pos_f.astype(jnp.int32)

    na = (jnp.sum(pc, axis=1, keepdims=True)) >> BT_SHIFT
    starts = off_i >> BT_SHIFT
    b_io = lax.broadcasted_iota(jnp.int32, (NB, 1), 0)
    b_eff = jnp.minimum(b_io, na - 1)
    ge = (b_eff >= starts).astype(jnp.int32)
    bexp = jnp.clip(jnp.sum(ge, axis=1, keepdims=True) - 1, 0, E - 1)
    bexp_ref[...] = jnp.concatenate([bexp, na], axis=0)


def _moe_body(bexp_ref, x_ref, w1_ref, b1_ref, w2_ref, b2_ref,
              g2_ref, be2_ref, o_ref):
    @pl.when(pl.program_id(0) < bexp_ref[NB])
    def _():
        xb = x_ref[...]
        h = _gelu_exact(_dot_nt(xb, w1_ref[0]) + b1_ref[0])
        z = _dot_nt(h, w2_ref[0]) + b2_ref[0]
        o_ref[...] = _ln(xb + z, g2_ref[...], be2_ref[...])


def _sc_mesh():
    return plsc.VectorSubcoreMesh(core_axis_name="c", subcore_axis_name="s")


def _dispatch_sc(y, pos):
    @functools.partial(
        pl.kernel, mesh=_sc_mesh(),
        out_type=jax.ShapeDtypeStruct((SPAD, D), jnp.float32),
        scratch_types=[
            pltpu.VMEM((BPW,), jnp.int32),
            pltpu.VMEM((BPW, D), jnp.float32),
            pltpu.SemaphoreType.DMA,
        ],
    )
    def k(y_hbm, pos_hbm, out_hbm, idx_v, rows_v, sem):
        wid = lax.axis_index("s") * 2 + lax.axis_index("c")
        base = wid * BPW
        pltpu.sync_copy(pos_hbm.at[pl.ds(base, BPW)], idx_v)
        pltpu.sync_copy(y_hbm.at[pl.ds(base, BPW)], rows_v)
        pltpu.async_copy(rows_v, out_hbm.at[idx_v], sem).wait()

    return k(y, pos)


def _combine_sc(zpad, pos):
    @functools.partial(
        pl.kernel, mesh=_sc_mesh(),
        out_type=jax.ShapeDtypeStruct((S, D), jnp.float32),
        scratch_types=[
            pltpu.VMEM((BPW,), jnp.int32),
            pltpu.VMEM((BPW, D), jnp.float32),
            pltpu.SemaphoreType.DMA,
        ],
    )
    def k(z_hbm, pos_hbm, out_hbm, idx_v, rows_v, sem):
        wid = lax.axis_index("s") * 2 + lax.axis_index("c")
        base = wid * BPW
        pltpu.sync_copy(pos_hbm.at[pl.ds(base, BPW)], idx_v)
        pltpu.async_copy(z_hbm.at[idx_v], rows_v, sem).wait()
        pltpu.sync_copy(rows_v, out_hbm.at[pl.ds(base, BPW)])

    return k(zpad, pos)


def kernel(x, Wqkv, bqkv, Wo, bo, Wg, bg, W1, b1, W2, b2, g1, be1, g2, be2):
    f32 = jnp.float32
    xf = x.reshape(S, D)

    qkv = pl.pallas_call(
        _qkv_body,
        grid=(6,),
        in_specs=[
            pl.BlockSpec((S, D), lambda n: (0, 0)),
            pl.BlockSpec((512, D), lambda n: (n, 0)),
            pl.BlockSpec((1, 512), lambda n: (0, n)),
        ],
        out_specs=pl.BlockSpec((S, 512), lambda n: (0, n)),
        out_shape=jax.ShapeDtypeStruct((S, 3 * D), f32),
    )(xf, Wqkv, bqkv.reshape(1, -1))

    qkv3 = qkv.reshape(S, 3 * H, DH).transpose(1, 0, 2)
    o3 = pl.pallas_call(
        _attn_body,
        grid=(H, S // BQ),
        in_specs=[
            pl.BlockSpec((1, BQ, DH), lambda h, qb: (h, qb, 0)),
            pl.BlockSpec((1, S, DH), lambda h, qb: (H + h, 0, 0)),
            pl.BlockSpec((1, S, DH), lambda h, qb: (2 * H + h, 0, 0)),
        ],
        out_specs=pl.BlockSpec((1, BQ, DH), lambda h, qb: (h, qb, 0)),
        out_shape=jax.ShapeDtypeStruct((H, S, DH), f32),
    )(qkv3, qkv3, qkv3)
    o_heads = o3.transpose(1, 0, 2).reshape(S, D)

    y, logits = pl.pallas_call(
        _post_attn_body,
        grid=(S // BQ,),
        in_specs=[
            pl.BlockSpec((BQ, D), lambda i: (i, 0)),
            pl.BlockSpec((BQ, D), lambda i: (i, 0)),
            pl.BlockSpec((D, D), lambda i: (0, 0)),
            pl.BlockSpec((1, D), lambda i: (0, 0)),
            pl.BlockSpec((1, D), lambda i: (0, 0)),
            pl.BlockSpec((1, D), lambda i: (0, 0)),
            pl.BlockSpec((E, D), lambda i: (0, 0)),
            pl.BlockSpec((1, E), lambda i: (0, 0)),
        ],
        out_specs=[
            pl.BlockSpec((BQ, D), lambda i: (i, 0)),
            pl.BlockSpec((BQ, E), lambda i: (i, 0)),
        ],
        out_shape=[
            jax.ShapeDtypeStruct((S, D), f32),
            jax.ShapeDtypeStruct((S, E), f32),
        ],
    )(o_heads, xf, Wo, bo.reshape(1, -1), g1.reshape(1, -1),
      be1.reshape(1, -1), Wg, bg.reshape(1, -1))

    pos2, bexp2 = pl.pallas_call(
        _route_body,
        out_shape=[
            jax.ShapeDtypeStruct((S, 1), jnp.int32),
            jax.ShapeDtypeStruct((NB + 1, 1), jnp.int32),
        ],
    )(logits)
    pos = pos2.reshape(S)
    block_expert = bexp2.reshape(NB + 1)

    xpad = _dispatch_sc(y, pos)

    zpad = pl.pallas_call(
        _moe_body,
        grid_spec=pltpu.PrefetchScalarGridSpec(
            num_scalar_prefetch=1,
            grid=(NB,),
            in_specs=[
                pl.BlockSpec((BT, D), lambda b, be: (jnp.minimum(b, be[NB] - 1), 0)),
                pl.BlockSpec((1, NHID, D), lambda b, be: (be[b], 0, 0)),
                pl.BlockSpec((1, 1, NHID), lambda b, be: (be[b], 0, 0)),
                pl.BlockSpec((1, D, NHID), lambda b, be: (be[b], 0, 0)),
                pl.BlockSpec((1, 1, D), lambda b, be: (be[b], 0, 0)),
                pl.BlockSpec((1, D), lambda b, be: (0, 0)),
                pl.BlockSpec((1, D), lambda b, be: (0, 0)),
            ],
            out_specs=pl.BlockSpec((BT, D),
                                   lambda b, be: (jnp.minimum(b, be[NB] - 1), 0)),
        ),
        out_shape=jax.ShapeDtypeStruct((SPAD, D), f32),
    )(block_expert, xpad, W1, b1.reshape(E, 1, NHID), W2,
      b2.reshape(E, 1, D), g2.reshape(1, -1), be2.reshape(1, -1))

    out = _combine_sc(zpad, pos)
    return out.reshape(1, S, D)


# speedup vs baseline: 1.1780x; 1.1780x over previous
import functools

import jax
import jax.numpy as jnp
from jax import lax
from jax.experimental import pallas as pl
from jax.experimental.pallas import tpu as pltpu
from jax.experimental.pallas import tpu_sc as plsc

D = 1024
H = 16
DH = 64
NHID = 2048
E = 8
S = 2048
EPS = 1e-5

BT = 256
BT_SHIFT = 8
NB = S // BT + E - 1
SPAD = NB * BT

BQ = 512

NW = 32
BPW = S // NW


def _gelu_exact(x):
    z = x * 0.7071067811865476
    a = jnp.abs(z)
    t = 1.0 / (1.0 + 0.3275911 * a)
    poly = t * (0.254829592 + t * (-0.284496736 + t * (1.421413741
               + t * (-1.453152027 + t * 1.061405429))))
    erf_abs = 1.0 - poly * jnp.exp(-a * a)
    erf = jnp.sign(z) * erf_abs
    return 0.5 * x * (1.0 + erf)


def _ln(r, g, b):
    mu = jnp.mean(r, axis=1, keepdims=True)
    var = jnp.mean((r - mu) ** 2, axis=1, keepdims=True)
    return (r - mu) * lax.rsqrt(var + EPS) * g + b


def _dot_nt(a, b):
    return lax.dot_general(a, b, (((1,), (1,)), ((), ())),
                           preferred_element_type=jnp.float32)


def _dot_nn(a, b):
    return lax.dot_general(a, b, (((1,), (0,)), ((), ())),
                           preferred_element_type=jnp.float32)


def _qkv_body(x_ref, w_ref, b_ref, o_ref):
    o_ref[...] = _dot_nt(x_ref[...], w_ref[...]) + b_ref[...]


_SCALE2 = 0.125 * 1.4426950408889634


def _attn_body(q_ref, k_ref, v_ref, o_ref):
    q = q_ref[0]
    k = k_ref[0]
    q2 = jnp.sum(q * q, axis=1, keepdims=True)
    k2m = jnp.max(jnp.sum(k * k, axis=1, keepdims=True))
    mhat = jnp.sqrt(q2 * k2m) * _SCALE2
    q_aug = jnp.concatenate([q * _SCALE2, mhat], axis=1)
    k_aug = jnp.concatenate(
        [k, jnp.full((S, 1), -1.0, jnp.float32)], axis=1)
    p = jnp.exp2(_dot_nt(q_aug, k_aug))
    v_aug = jnp.concatenate(
        [v_ref[0], jnp.ones((S, 1), jnp.float32)], axis=1)
    od = _dot_nn(p, v_aug)
    o_ref[0] = od[:, :DH] / od[:, DH:]


def _post_attn_body(o_ref, x_ref, wo_ref, bo_ref, g1_ref, be1_ref,
                    wg_ref, bg_ref, y_ref, logits_ref):
    att = _dot_nt(o_ref[...], wo_ref[...]) + bo_ref[...]
    y = _ln(x_ref[...] + att, g1_ref[...], be1_ref[...])
    y_ref[...] = y
    logits_ref[...] = _dot_nt(y, wg_ref[...]) + bg_ref[...]


def _route_body(logits_ref, pos_ref, bexp_ref):
    logits = logits_ref[...]
    m = jnp.max(logits, axis=1, keepdims=True)
    iota_e = lax.broadcasted_iota(jnp.int32, (S, E), 1)
    idxm = jnp.where(logits == m, iota_e, E)
    idx = jnp.min(idxm, axis=1, keepdims=True)
    onehot = (iota_e == idx).astype(jnp.float32)

    r_io = lax.broadcasted_iota(jnp.int32, (S, S), 0)
    c_io = lax.broadcasted_iota(jnp.int32, (S, S), 1)
    tril = (r_io > c_io).astype(jnp.bfloat16)
    ranks_all = lax.dot_general(tril, onehot.astype(jnp.bfloat16),
                                (((1,), (0,)), ((), ())),
                                preferred_element_type=jnp.float32)

    counts = jnp.sum(onehot, axis=0, keepdims=True)
    counts_i = counts.astype(jnp.int32)
    pc = ((counts_i + (BT - 1)) >> BT_SHIFT) << BT_SHIFT
    e_r = lax.broadcasted_iota(jnp.int32, (E, E), 0)
    e_c = lax.broadcasted_iota(jnp.int32, (E, E), 1)
    tri8 = (e_r < e_c).astype(jnp.float32)
    off = _dot_nn(pc.astype(jnp.float32), tri8)
    off_i = off.astype(jnp.int32)

    pos_f = jnp.sum((ranks_all + off) * onehot, axis=1, keepdims=True)
    pos_ref[...] = pos_f.astype(jnp.int32)

    na = (jnp.sum(pc, axis=1, keepdims=True)) >> BT_SHIFT
    starts = off_i >> BT_SHIFT
    b_io = lax.broadcasted_iota(jnp.int32, (NB, 1), 0)
    b_eff = jnp.minimum(b_io, na - 1)
    ge = (b_eff >= starts).astype(jnp.int32)
    bexp = jnp.clip(jnp.sum(ge, axis=1, keepdims=True) - 1, 0, E - 1)
    bexp_ref[...] = jnp.concatenate([bexp, na], axis=0)


def _moe_body(bexp_ref, x_ref, w1_ref, b1_ref, w2_ref, b2_ref,
              g2_ref, be2_ref, o_ref):
    @pl.when(pl.program_id(0) < bexp_ref[NB])
    def _():
        xb = x_ref[...]
        h = _gelu_exact(_dot_nt(xb, w1_ref[0]) + b1_ref[0])
        z = _dot_nt(h, w2_ref[0]) + b2_ref[0]
        o_ref[...] = _ln(xb + z, g2_ref[...], be2_ref[...])


def _sc_mesh():
    return plsc.VectorSubcoreMesh(core_axis_name="c", subcore_axis_name="s")


def _dispatch_sc(y, pos):
    @functools.partial(
        pl.kernel, mesh=_sc_mesh(),
        out_type=jax.ShapeDtypeStruct((SPAD, D), jnp.float32),
        scratch_types=[
            pltpu.VMEM((BPW,), jnp.int32),
            pltpu.VMEM((BPW, D), jnp.float32),
            pltpu.SemaphoreType.DMA,
        ],
    )
    def k(y_hbm, pos_hbm, out_hbm, idx_v, rows_v, sem):
        wid = lax.axis_index("s") * 2 + lax.axis_index("c")
        base = wid * BPW
        pltpu.sync_copy(pos_hbm.at[pl.ds(base, BPW)], idx_v)
        pltpu.sync_copy(y_hbm.at[pl.ds(base, BPW)], rows_v)
        pltpu.async_copy(rows_v, out_hbm.at[idx_v], sem).wait()

    return k(y, pos)


def _combine_sc(zpad, pos):
    @functools.partial(
        pl.kernel, mesh=_sc_mesh(),
        out_type=jax.ShapeDtypeStruct((S, D), jnp.float32),
        scratch_types=[
            pltpu.VMEM((BPW,), jnp.int32),
            pltpu.VMEM((BPW, D), jnp.float32),
            pltpu.SemaphoreType.DMA,
        ],
    )
    def k(z_hbm, pos_hbm, out_hbm, idx_v, rows_v, sem):
        wid = lax.axis_index("s") * 2 + lax.axis_index("c")
        base = wid * BPW
        pltpu.sync_copy(pos_hbm.at[pl.ds(base, BPW)], idx_v)
        pltpu.async_copy(z_hbm.at[idx_v], rows_v, sem).wait()
        pltpu.sync_copy(rows_v, out_hbm.at[pl.ds(base, BPW)])

    return k(zpad, pos)


def kernel(x, Wqkv, bqkv, Wo, bo, Wg, bg, W1, b1, W2, b2, g1, be1, g2, be2):
    f32 = jnp.float32
    xf = x.reshape(S, D)

    qkv = pl.pallas_call(
        _qkv_body,
        grid=(6,),
        in_specs=[
            pl.BlockSpec((S, D), lambda n: (0, 0)),
            pl.BlockSpec((512, D), lambda n: (n, 0)),
            pl.BlockSpec((1, 512), lambda n: (0, n)),
        ],
        out_specs=pl.BlockSpec((S, 512), lambda n: (0, n)),
        out_shape=jax.ShapeDtypeStruct((S, 3 * D), f32),
    )(xf, Wqkv, bqkv.reshape(1, -1))

    qkv3 = qkv.reshape(S, 3 * H, DH).transpose(1, 0, 2)
    o3 = pl.pallas_call(
        _attn_body,
        grid=(H, S // BQ),
        in_specs=[
            pl.BlockSpec((1, BQ, DH), lambda h, qb: (h, qb, 0)),
            pl.BlockSpec((1, S, DH), lambda h, qb: (H + h, 0, 0)),
            pl.BlockSpec((1, S, DH), lambda h, qb: (2 * H + h, 0, 0)),
        ],
        out_specs=pl.BlockSpec((1, BQ, DH), lambda h, qb: (h, qb, 0)),
        out_shape=jax.ShapeDtypeStruct((H, S, DH), f32),
    )(qkv3, qkv3, qkv3)
    o_heads = o3.transpose(1, 0, 2).reshape(S, D)

    y, logits = pl.pallas_call(
        _post_attn_body,
        grid=(S // BQ,),
        in_specs=[
            pl.BlockSpec((BQ, D), lambda i: (i, 0)),
            pl.BlockSpec((BQ, D), lambda i: (i, 0)),
            pl.BlockSpec((D, D), lambda i: (0, 0)),
            pl.BlockSpec((1, D), lambda i: (0, 0)),
            pl.BlockSpec((1, D), lambda i: (0, 0)),
            pl.BlockSpec((1, D), lambda i: (0, 0)),
            pl.BlockSpec((E, D), lambda i: (0, 0)),
            pl.BlockSpec((1, E), lambda i: (0, 0)),
        ],
        out_specs=[
            pl.BlockSpec((BQ, D), lambda i: (i, 0)),
            pl.BlockSpec((BQ, E), lambda i: (i, 0)),
        ],
        out_shape=[
            jax.ShapeDtypeStruct((S, D), f32),
            jax.ShapeDtypeStruct((S, E), f32),
        ],
    )(o_heads, xf, Wo, bo.reshape(1, -1), g1.reshape(1, -1),
      be1.reshape(1, -1), Wg, bg.reshape(1, -1))

    pos2, bexp2 = pl.pallas_call(
        _route_body,
        out_shape=[
            jax.ShapeDtypeStruct((S, 1), jnp.int32),
            jax.ShapeDtypeStruct((NB + 1, 1), jnp.int32),
        ],
    )(logits)
    pos = pos2.reshape(S)
    block_expert = bexp2.reshape(NB + 1)

    xpad = _dispatch_sc(y, pos)

    zpad = pl.pallas_call(
        _moe_body,
        grid_spec=pltpu.PrefetchScalarGridSpec(
            num_scalar_prefetch=1,
            grid=(NB,),
            in_specs=[
                pl.BlockSpec((BT, D), lambda b, be: (jnp.minimum(b, be[NB] - 1), 0)),
                pl.BlockSpec((1, NHID, D), lambda b, be: (be[b], 0, 0)),
                pl.BlockSpec((1, 1, NHID), lambda b, be: (be[b], 0, 0)),
                pl.BlockSpec((1, D, NHID), lambda b, be: (be[b], 0, 0)),
                pl.BlockSpec((1, 1, D), lambda b, be: (be[b], 0, 0)),
                pl.BlockSpec((1, D), lambda b, be: (0, 0)),
                pl.BlockSpec((1, D), lambda b, be: (0, 0)),
            ],
            out_specs=pl.BlockSpec((BT, D),
                                   lambda b, be: (jnp.minimum(b, be[NB] - 1), 0)),
        ),
        out_shape=jax.ShapeDtypeStruct((SPAD, D), f32),
    )(block_expert, xpad, W1, b1.reshape(E, 1, NHID), W2,
      b2.reshape(E, 1, D), g2.reshape(1, -1), be2.reshape(1, -1))

    out = _combine_sc(zpad, pos)
    return out.reshape(1, S, D)
